# Initial kernel scaffold; baseline (speedup 1.0000x reference)
import jax
import jax.numpy as jnp
from jax.experimental import pallas as pl


def kernel(inputs, conv_1_w, conv_1_b, conv_2_w, conv_2_b, pre_vq_w, pre_vq_b, res0_w3, res0_w1, res1_w3, res1_w1, conv_3_w, conv_3_b):
    raise NotImplementedError("write your pallas kernel here")


# speedup vs baseline: 24.3617x; 24.3617x over previous
import jax
import jax.numpy as jnp
from jax.experimental import pallas as pl
from jax.experimental.pallas import tpu as pltpu

_BF16 = jnp.bfloat16
_F32 = jnp.float32


def _shift_r(v):
    z = jnp.zeros((v.shape[0], 1), v.dtype)
    return jnp.concatenate([z, v[:, :-1]], axis=1)


def _shift_l(v):
    z = jnp.zeros((v.shape[0], 1), v.dtype)
    return jnp.concatenate([v[:, 1:], z], axis=1)


def _make_body(c1, num_res):
    def body(x4_ref, w1_ref, b1_ref, w2_ref, b2_ref, wh_ref, bh_ref,
             w3s_ref, w1s_ref, wp_ref, bp_ref, o_ref):
        x4 = x4_ref[...]
        taps1 = jnp.concatenate([_shift_r(x4), x4, _shift_l(x4)], axis=0)
        y = jnp.dot(w1_ref[...], taps1, preferred_element_type=_F32)
        y01 = jnp.maximum(y + b1_ref[...], 0.0).astype(_BF16)

        taps2 = jnp.concatenate(
            [y01, _shift_r(y01[c1:]), _shift_l(y01[:c1])], axis=0)
        h = jnp.dot(w2_ref[...], taps2, preferred_element_type=_F32)
        v = jnp.maximum(h + b2_ref[...], 0.0).astype(_BF16)

        def conv3(vb, wcat):
            taps = jnp.concatenate([_shift_r(vb), vb, _shift_l(vb)], axis=0)
            return jnp.dot(wcat, taps, preferred_element_type=_F32)

        x = conv3(v, wh_ref[...]) + bh_ref[...]

        for l in range(num_res):
            vb = jnp.maximum(x, 0.0).astype(_BF16)
            hh = jnp.maximum(conv3(vb, w3s_ref[l]), 0.0).astype(_BF16)
            x = x + jnp.dot(w1s_ref[l], hh, preferred_element_type=_F32)

        vb = jnp.maximum(x, 0.0).astype(_BF16)
        o_ref[...] = (jnp.dot(wp_ref[...], vb, preferred_element_type=_F32)
                      + bp_ref[...])[None]
    return body


def kernel(inputs, conv_1_w, conv_1_b, conv_2_w, conv_2_b, pre_vq_w, pre_vq_b,
           res0_w3, res0_w1, res1_w3, res1_w1, conv_3_w, conv_3_b):
    B, L = inputs.shape
    c1 = conv_1_w.shape[0]
    nh = conv_2_w.shape[0]
    rh = res0_w3.shape[0]
    e = pre_vq_w.shape[0]
    lf = L // 4
    m = B * lf

    x4 = inputs.reshape(B, lf, 4).transpose(2, 0, 1).reshape(4, m).astype(_BF16)

    w1 = conv_1_w.astype(_F32)[:, 0, :]
    w1cat = jnp.zeros((2 * c1, 12), _F32)
    w1cat = w1cat.at[:c1, 3:7].set(w1)
    w1cat = w1cat.at[c1:, 5:9].set(w1)
    w1cat = w1cat.astype(_BF16)
    b1 = jnp.concatenate([conv_1_b, conv_1_b]).reshape(2 * c1, 1).astype(_F32)

    w2 = conv_2_w.astype(_F32)
    w2cat = jnp.concatenate(
        [w2[:, :, 1], w2[:, :, 2], w2[:, :, 0], w2[:, :, 3]],
        axis=1).astype(_BF16)
    b2 = conv_2_b.reshape(nh, 1).astype(_F32)

    def cat3(w):
        return jnp.concatenate([w[:, :, 0], w[:, :, 1], w[:, :, 2]], axis=1)

    wh = cat3(conv_3_w.astype(_F32)).astype(_BF16)
    bh = conv_3_b.reshape(nh, 1).astype(_F32)
    w3s = jnp.stack([cat3(res0_w3.astype(_F32)),
                     cat3(res1_w3.astype(_F32))]).astype(_BF16)
    w1s = jnp.stack([res0_w1[:, :, 0],
                     res1_w1[:, :, 0]]).astype(_BF16)
    wp = pre_vq_w[:, :, 0].astype(_BF16)
    bp = pre_vq_b.reshape(e, 1).astype(_F32)

    out = pl.pallas_call(
        _make_body(c1, 2),
        out_shape=jax.ShapeDtypeStruct((B, e, lf), _F32),
        grid_spec=pltpu.PrefetchScalarGridSpec(
            num_scalar_prefetch=0,
            grid=(B,),
            in_specs=[
                pl.BlockSpec((4, lf), lambda i: (0, i)),
                pl.BlockSpec((2 * c1, 12), lambda i: (0, 0)),
                pl.BlockSpec((2 * c1, 1), lambda i: (0, 0)),
                pl.BlockSpec((nh, 4 * c1), lambda i: (0, 0)),
                pl.BlockSpec((nh, 1), lambda i: (0, 0)),
                pl.BlockSpec((nh, 3 * nh), lambda i: (0, 0)),
                pl.BlockSpec((nh, 1), lambda i: (0, 0)),
                pl.BlockSpec((2, rh, 3 * nh), lambda i: (0, 0, 0)),
                pl.BlockSpec((2, nh, rh), lambda i: (0, 0, 0)),
                pl.BlockSpec((e, nh), lambda i: (0, 0)),
                pl.BlockSpec((e, 1), lambda i: (0, 0)),
            ],
            out_specs=pl.BlockSpec((1, e, lf), lambda i: (i, 0, 0)),
        ),
        compiler_params=pltpu.CompilerParams(
            dimension_semantics=("parallel",),
            vmem_limit_bytes=64 * 1024 * 1024,
        ),
    )(x4, w1cat, b1, w2cat, b2, wh, bh, w3s, w1s, wp, bp)
    return out


# speedup vs baseline: 29.6605x; 1.2175x over previous
import jax
import jax.numpy as jnp
from jax.experimental import pallas as pl
from jax.experimental.pallas import tpu as pltpu

_BF16 = jnp.bfloat16
_F32 = jnp.float32


def _shift_r(v):
    z = jnp.zeros((v.shape[0], 1), v.dtype)
    return jnp.concatenate([z, v[:, :-1]], axis=1)


def _shift_l(v):
    z = jnp.zeros((v.shape[0], 1), v.dtype)
    return jnp.concatenate([v[:, 1:], z], axis=1)


def _make_body(c1, rh, num_res):
    def body(x4_ref, w1_ref, w2_ref, b2_ref, wh_ref,
             w3s_ref, w1s_ref, wp_ref, bp_ref, o_ref):
        x4 = x4_ref[...]
        ones = jnp.ones((1, x4.shape[1]), _BF16)
        taps1 = jnp.concatenate([_shift_r(x4), x4, _shift_l(x4), ones],
                                axis=0)
        y = jnp.dot(w1_ref[...], taps1, preferred_element_type=_F32)
        y01 = jnp.maximum(y.astype(_BF16), 0)

        taps2 = jnp.concatenate(
            [y01, _shift_r(y01[c1:]), _shift_l(y01[:c1])], axis=0)
        h = jnp.dot(w2_ref[...], taps2, preferred_element_type=_F32)
        v = jnp.maximum((h + b2_ref[...]).astype(_BF16), 0)

        tapsh = jnp.concatenate([_shift_r(v), v, _shift_l(v), ones], axis=0)
        x = jnp.dot(wh_ref[...], tapsh, preferred_element_type=_F32)

        for l in range(num_res):
            vb = jnp.maximum(x.astype(_BF16), 0)
            abc = jnp.dot(w3s_ref[l], vb, preferred_element_type=_F32)
            hsum = (_shift_r(abc[:rh]) + abc[rh:2 * rh]
                    + _shift_l(abc[2 * rh:]))
            hh = jnp.maximum(hsum.astype(_BF16), 0)
            x = x + jnp.dot(w1s_ref[l], hh, preferred_element_type=_F32)

        vb = jnp.maximum(x.astype(_BF16), 0)
        o_ref[...] = (jnp.dot(wp_ref[...], vb, preferred_element_type=_F32)
                      + bp_ref[...])[None]
    return body


def kernel(inputs, conv_1_w, conv_1_b, conv_2_w, conv_2_b, pre_vq_w, pre_vq_b,
           res0_w3, res0_w1, res1_w3, res1_w1, conv_3_w, conv_3_b):
    B, L = inputs.shape
    c1 = conv_1_w.shape[0]
    nh = conv_2_w.shape[0]
    rh = res0_w3.shape[0]
    e = pre_vq_w.shape[0]
    lf = L // 4
    m = B * lf

    x4 = inputs.reshape(B, lf, 4).transpose(2, 0, 1).reshape(4, m).astype(_BF16)

    w1 = conv_1_w.astype(_F32)[:, 0, :]
    w1cat = jnp.zeros((2 * c1, 13), _F32)
    w1cat = w1cat.at[:c1, 3:7].set(w1)
    w1cat = w1cat.at[c1:, 5:9].set(w1)
    w1cat = w1cat.at[:c1, 12].set(conv_1_b.astype(_F32))
    w1cat = w1cat.at[c1:, 12].set(conv_1_b.astype(_F32))
    w1cat = w1cat.astype(_BF16)

    w2 = conv_2_w.astype(_F32)
    w2cat = jnp.concatenate(
        [w2[:, :, 1], w2[:, :, 2], w2[:, :, 0], w2[:, :, 3]],
        axis=1).astype(_BF16)
    b2 = conv_2_b.reshape(nh, 1).astype(_F32)

    w3h = conv_3_w.astype(_F32)
    wh = jnp.concatenate(
        [w3h[:, :, 0], w3h[:, :, 1], w3h[:, :, 2],
         conv_3_b.reshape(nh, 1).astype(_F32)], axis=1).astype(_BF16)
    w3s = jnp.stack(
        [jnp.concatenate([w[:, :, 0], w[:, :, 1], w[:, :, 2]], axis=0)
         for w in (res0_w3.astype(_F32), res1_w3.astype(_F32))]
    ).astype(_BF16)
    w1s = jnp.stack([res0_w1[:, :, 0],
                     res1_w1[:, :, 0]]).astype(_BF16)
    wp = pre_vq_w[:, :, 0].astype(_BF16)
    bp = pre_vq_b.reshape(e, 1).astype(_F32)

    out = pl.pallas_call(
        _make_body(c1, rh, 2),
        out_shape=jax.ShapeDtypeStruct((B, e, lf), _F32),
        grid_spec=pltpu.PrefetchScalarGridSpec(
            num_scalar_prefetch=0,
            grid=(B,),
            in_specs=[
                pl.BlockSpec((4, lf), lambda i: (0, i)),
                pl.BlockSpec((2 * c1, 13), lambda i: (0, 0)),
                pl.BlockSpec((nh, 4 * c1), lambda i: (0, 0)),
                pl.BlockSpec((nh, 1), lambda i: (0, 0)),
                pl.BlockSpec((nh, 3 * nh + 1), lambda i: (0, 0)),
                pl.BlockSpec((2, 3 * rh, nh), lambda i: (0, 0, 0)),
                pl.BlockSpec((2, nh, rh), lambda i: (0, 0, 0)),
                pl.BlockSpec((e, nh), lambda i: (0, 0)),
                pl.BlockSpec((e, 1), lambda i: (0, 0)),
            ],
            out_specs=pl.BlockSpec((1, e, lf), lambda i: (i, 0, 0)),
        ),
        compiler_params=pltpu.CompilerParams(
            dimension_semantics=("parallel",),
            vmem_limit_bytes=64 * 1024 * 1024,
        ),
    )(x4, w1cat, w2cat, b2, wh, w3s, w1s, wp, bp)
    return out


# speedup vs baseline: 32.2543x; 1.0874x over previous
import jax
import jax.numpy as jnp
from jax.experimental import pallas as pl
from jax.experimental.pallas import tpu as pltpu

_BF16 = jnp.bfloat16
_F32 = jnp.float32


def _make_shifts(g, lf):
    def shift_r(v):
        z = jnp.zeros((v.shape[0], 1), v.dtype)
        pieces = []
        for s in range(g):
            pieces += [z, v[:, s * lf:(s + 1) * lf - 1]]
        return jnp.concatenate(pieces, axis=1)

    def shift_l(v):
        z = jnp.zeros((v.shape[0], 1), v.dtype)
        pieces = []
        for s in range(g):
            pieces += [v[:, s * lf + 1:(s + 1) * lf], z]
        return jnp.concatenate(pieces, axis=1)

    return shift_r, shift_l


def _make_body(c1, rh, num_res, g, lf):
    _shift_r, _shift_l = _make_shifts(g, lf)

    def body(x4_ref, w1_ref, w2_ref, b2_ref, wh_ref,
             w3s_ref, w1s_ref, wp_ref, bp_ref, o_ref):
        x4 = x4_ref[...]
        ones = jnp.ones((1, x4.shape[1]), _BF16)
        taps1 = jnp.concatenate([_shift_r(x4), x4, _shift_l(x4), ones],
                                axis=0)
        y = jnp.dot(w1_ref[...], taps1, preferred_element_type=_F32)
        y01 = jnp.maximum(y.astype(_BF16), 0)

        taps2 = jnp.concatenate(
            [y01, _shift_r(y01[c1:]), _shift_l(y01[:c1])], axis=0)
        h = jnp.dot(w2_ref[...], taps2, preferred_element_type=_F32)
        v = jnp.maximum((h + b2_ref[...]).astype(_BF16), 0)

        tapsh = jnp.concatenate([_shift_r(v), v, _shift_l(v), ones], axis=0)
        x = jnp.dot(wh_ref[...], tapsh, preferred_element_type=_F32)

        for l in range(num_res):
            vb = jnp.maximum(x.astype(_BF16), 0)
            abc = jnp.dot(w3s_ref[l], vb, preferred_element_type=_F32)
            hsum = (_shift_r(abc[:rh]) + abc[rh:2 * rh]
                    + _shift_l(abc[2 * rh:]))
            hh = jnp.maximum(hsum.astype(_BF16), 0)
            x = x + jnp.dot(w1s_ref[l], hh, preferred_element_type=_F32)

        vb = jnp.maximum(x.astype(_BF16), 0)
        yout = jnp.dot(wp_ref[...], vb, preferred_element_type=_F32) + bp_ref[...]
        for s in range(g):
            o_ref[s] = yout[:, s * lf:(s + 1) * lf]
    return body


def kernel(inputs, conv_1_w, conv_1_b, conv_2_w, conv_2_b, pre_vq_w, pre_vq_b,
           res0_w3, res0_w1, res1_w3, res1_w1, conv_3_w, conv_3_b):
    B, L = inputs.shape
    c1 = conv_1_w.shape[0]
    nh = conv_2_w.shape[0]
    rh = res0_w3.shape[0]
    e = pre_vq_w.shape[0]
    lf = L // 4
    m = B * lf

    x4 = inputs.reshape(B, lf, 4).transpose(2, 0, 1).reshape(4, m).astype(_BF16)

    w1 = conv_1_w.astype(_F32)[:, 0, :]
    w1cat = jnp.zeros((2 * c1, 13), _F32)
    w1cat = w1cat.at[:c1, 3:7].set(w1)
    w1cat = w1cat.at[c1:, 5:9].set(w1)
    w1cat = w1cat.at[:c1, 12].set(conv_1_b.astype(_F32))
    w1cat = w1cat.at[c1:, 12].set(conv_1_b.astype(_F32))
    w1cat = w1cat.astype(_BF16)

    w2 = conv_2_w.astype(_F32)
    w2cat = jnp.concatenate(
        [w2[:, :, 1], w2[:, :, 2], w2[:, :, 0], w2[:, :, 3]],
        axis=1).astype(_BF16)
    b2 = conv_2_b.reshape(nh, 1).astype(_F32)

    w3h = conv_3_w.astype(_F32)
    wh = jnp.concatenate(
        [w3h[:, :, 0], w3h[:, :, 1], w3h[:, :, 2],
         conv_3_b.reshape(nh, 1).astype(_F32)], axis=1).astype(_BF16)
    w3s = jnp.stack(
        [jnp.concatenate([w[:, :, 0], w[:, :, 1], w[:, :, 2]], axis=0)
         for w in (res0_w3.astype(_F32), res1_w3.astype(_F32))]
    ).astype(_BF16)
    w1s = jnp.stack([res0_w1[:, :, 0],
                     res1_w1[:, :, 0]]).astype(_BF16)
    wp = pre_vq_w[:, :, 0].astype(_BF16)
    bp = pre_vq_b.reshape(e, 1).astype(_F32)

    g = 2
    out = pl.pallas_call(
        _make_body(c1, rh, 2, g, lf),
        out_shape=jax.ShapeDtypeStruct((B, e, lf), _F32),
        grid_spec=pltpu.PrefetchScalarGridSpec(
            num_scalar_prefetch=0,
            grid=(B // g,),
            in_specs=[
                pl.BlockSpec((4, g * lf), lambda i: (0, i)),
                pl.BlockSpec((2 * c1, 13), lambda i: (0, 0)),
                pl.BlockSpec((nh, 4 * c1), lambda i: (0, 0)),
                pl.BlockSpec((nh, 1), lambda i: (0, 0)),
                pl.BlockSpec((nh, 3 * nh + 1), lambda i: (0, 0)),
                pl.BlockSpec((2, 3 * rh, nh), lambda i: (0, 0, 0)),
                pl.BlockSpec((2, nh, rh), lambda i: (0, 0, 0)),
                pl.BlockSpec((e, nh), lambda i: (0, 0)),
                pl.BlockSpec((e, 1), lambda i: (0, 0)),
            ],
            out_specs=pl.BlockSpec((g, e, lf), lambda i: (i, 0, 0)),
        ),
        compiler_params=pltpu.CompilerParams(
            dimension_semantics=("parallel",),
            vmem_limit_bytes=64 * 1024 * 1024,
        ),
    )(x4, w1cat, w2cat, b2, wh, w3s, w1s, wp, bp)
    return out


# speedup vs baseline: 32.3344x; 1.0025x over previous
import jax
import jax.numpy as jnp
from jax.experimental import pallas as pl
from jax.experimental.pallas import tpu as pltpu

_BF16 = jnp.bfloat16
_F32 = jnp.float32


def _make_shifts(g, lf):
    def shift_r(v):
        z = jnp.zeros((v.shape[0], 1), v.dtype)
        pieces = []
        for s in range(g):
            pieces += [z, v[:, s * lf:(s + 1) * lf - 1]]
        return jnp.concatenate(pieces, axis=1)

    def shift_l(v):
        z = jnp.zeros((v.shape[0], 1), v.dtype)
        pieces = []
        for s in range(g):
            pieces += [v[:, s * lf + 1:(s + 1) * lf], z]
        return jnp.concatenate(pieces, axis=1)

    return shift_r, shift_l


def _make_body(c1, rh, num_res, g, lf):
    _shift_r, _shift_l = _make_shifts(g, lf)

    def body(x4_ref, w1_ref, w2_ref, b2_ref, wh_ref,
             w3s_ref, w1s_ref, wp_ref, bp_ref, o_ref):
        x4 = x4_ref[...]
        ones = jnp.ones((1, x4.shape[1]), _BF16)
        taps1 = jnp.concatenate([_shift_r(x4), x4, _shift_l(x4), ones],
                                axis=0)
        y = jnp.dot(w1_ref[...], taps1, preferred_element_type=_F32)
        y01 = jnp.maximum(y.astype(_BF16), 0)

        taps2 = jnp.concatenate(
            [y01, _shift_r(y01[c1:]), _shift_l(y01[:c1])], axis=0)
        h = jnp.dot(w2_ref[...], taps2, preferred_element_type=_F32)
        v = jnp.maximum((h + b2_ref[...]).astype(_BF16), 0)

        tapsh = jnp.concatenate([_shift_r(v), v, _shift_l(v), ones], axis=0)
        x = jnp.dot(wh_ref[...], tapsh, preferred_element_type=_F32)

        for l in range(num_res):
            vb = jnp.maximum(x.astype(_BF16), 0)
            abc = jnp.dot(w3s_ref[l], vb, preferred_element_type=_F32)
            hsum = (_shift_r(abc[:rh]) + abc[rh:2 * rh]
                    + _shift_l(abc[2 * rh:]))
            hh = jnp.maximum(hsum.astype(_BF16), 0)
            x = x + jnp.dot(w1s_ref[l], hh, preferred_element_type=_F32)

        vb = jnp.maximum(x.astype(_BF16), 0)
        yout = jnp.dot(wp_ref[...], vb, preferred_element_type=_F32) + bp_ref[...]
        for s in range(g):
            o_ref[s] = yout[:, s * lf:(s + 1) * lf]
    return body


def kernel(inputs, conv_1_w, conv_1_b, conv_2_w, conv_2_b, pre_vq_w, pre_vq_b,
           res0_w3, res0_w1, res1_w3, res1_w1, conv_3_w, conv_3_b):
    B, L = inputs.shape
    c1 = conv_1_w.shape[0]
    nh = conv_2_w.shape[0]
    rh = res0_w3.shape[0]
    e = pre_vq_w.shape[0]
    lf = L // 4
    m = B * lf

    x4 = inputs.reshape(B, lf, 4).transpose(2, 0, 1).reshape(4, m).astype(_BF16)

    w1 = conv_1_w.astype(_F32)[:, 0, :]
    w1cat = jnp.zeros((2 * c1, 13), _F32)
    w1cat = w1cat.at[:c1, 3:7].set(w1)
    w1cat = w1cat.at[c1:, 5:9].set(w1)
    w1cat = w1cat.at[:c1, 12].set(conv_1_b.astype(_F32))
    w1cat = w1cat.at[c1:, 12].set(conv_1_b.astype(_F32))
    w1cat = w1cat.astype(_BF16)

    w2 = conv_2_w.astype(_F32)
    w2cat = jnp.concatenate(
        [w2[:, :, 1], w2[:, :, 2], w2[:, :, 0], w2[:, :, 3]],
        axis=1).astype(_BF16)
    b2 = conv_2_b.reshape(nh, 1).astype(_F32)

    w3h = conv_3_w.astype(_F32)
    wh = jnp.concatenate(
        [w3h[:, :, 0], w3h[:, :, 1], w3h[:, :, 2],
         conv_3_b.reshape(nh, 1).astype(_F32)], axis=1).astype(_BF16)
    w3s = jnp.stack(
        [jnp.concatenate([w[:, :, 0], w[:, :, 1], w[:, :, 2]], axis=0)
         for w in (res0_w3.astype(_F32), res1_w3.astype(_F32))]
    ).astype(_BF16)
    w1s = jnp.stack([res0_w1[:, :, 0],
                     res1_w1[:, :, 0]]).astype(_BF16)
    wp = pre_vq_w[:, :, 0].astype(_BF16)
    bp = pre_vq_b.reshape(e, 1).astype(_F32)

    g = 4
    out = pl.pallas_call(
        _make_body(c1, rh, 2, g, lf),
        out_shape=jax.ShapeDtypeStruct((B, e, lf), _F32),
        grid_spec=pltpu.PrefetchScalarGridSpec(
            num_scalar_prefetch=0,
            grid=(B // g,),
            in_specs=[
                pl.BlockSpec((4, g * lf), lambda i: (0, i)),
                pl.BlockSpec((2 * c1, 13), lambda i: (0, 0)),
                pl.BlockSpec((nh, 4 * c1), lambda i: (0, 0)),
                pl.BlockSpec((nh, 1), lambda i: (0, 0)),
                pl.BlockSpec((nh, 3 * nh + 1), lambda i: (0, 0)),
                pl.BlockSpec((2, 3 * rh, nh), lambda i: (0, 0, 0)),
                pl.BlockSpec((2, nh, rh), lambda i: (0, 0, 0)),
                pl.BlockSpec((e, nh), lambda i: (0, 0)),
                pl.BlockSpec((e, 1), lambda i: (0, 0)),
            ],
            out_specs=pl.BlockSpec((g, e, lf), lambda i: (i, 0, 0)),
        ),
        compiler_params=pltpu.CompilerParams(
            dimension_semantics=("parallel",),
            vmem_limit_bytes=64 * 1024 * 1024,
        ),
    )(x4, w1cat, w2cat, b2, wh, w3s, w1s, wp, bp)
    return out


# speedup vs baseline: 33.1563x; 1.0254x over previous
import jax
import jax.numpy as jnp
from jax.experimental import pallas as pl
from jax.experimental.pallas import tpu as pltpu

_BF16 = jnp.bfloat16
_F32 = jnp.float32


def _make_shifts(g, lf):
    def shift_r(v):
        z = jnp.zeros((v.shape[0], 1), v.dtype)
        pieces = []
        for s in range(g):
            pieces += [z, v[:, s * lf:(s + 1) * lf - 1]]
        return jnp.concatenate(pieces, axis=1)

    def shift_l(v):
        z = jnp.zeros((v.shape[0], 1), v.dtype)
        pieces = []
        for s in range(g):
            pieces += [v[:, s * lf + 1:(s + 1) * lf], z]
        return jnp.concatenate(pieces, axis=1)

    return shift_r, shift_l


def _make_body(c1, rh, num_res, g, lf):
    _shift_r, _shift_l = _make_shifts(g, lf)

    def body(x4_ref, w1_ref, w2_ref, b2_ref, wh_ref,
             w3s_ref, w1s_ref, wp_ref, bp_ref, o_ref):
        x4 = x4_ref[...]
        ones = jnp.ones((1, x4.shape[1]), _BF16)
        taps1 = jnp.concatenate([_shift_r(x4), x4, _shift_l(x4), ones],
                                axis=0)
        y = jnp.dot(w1_ref[...], taps1, preferred_element_type=_F32)
        y01 = jnp.maximum(y.astype(_BF16), 0)

        taps2 = jnp.concatenate(
            [y01, _shift_r(y01[c1:]), _shift_l(y01[:c1])], axis=0)
        h = jnp.dot(w2_ref[...], taps2, preferred_element_type=_F32)
        v = jnp.maximum(h.astype(_BF16) + b2_ref[...], 0)

        tapsh = jnp.concatenate([_shift_r(v), v, _shift_l(v), ones], axis=0)
        x = jnp.dot(wh_ref[...], tapsh, preferred_element_type=_F32)

        for l in range(num_res):
            vb = jnp.maximum(x.astype(_BF16), 0)
            abc = jnp.dot(w3s_ref[l], vb, preferred_element_type=_F32)
            hsum = (_shift_r(abc[:rh]) + abc[rh:2 * rh]
                    + _shift_l(abc[2 * rh:]))
            hh = jnp.maximum(hsum.astype(_BF16), 0)
            x = x + jnp.dot(w1s_ref[l], hh, preferred_element_type=_F32)

        vb = jnp.maximum(x.astype(_BF16), 0)
        yout = jnp.dot(wp_ref[...], vb, preferred_element_type=_F32) + bp_ref[...]
        for s in range(g):
            o_ref[s] = yout[:, s * lf:(s + 1) * lf]
    return body


def kernel(inputs, conv_1_w, conv_1_b, conv_2_w, conv_2_b, pre_vq_w, pre_vq_b,
           res0_w3, res0_w1, res1_w3, res1_w1, conv_3_w, conv_3_b):
    B, L = inputs.shape
    c1 = conv_1_w.shape[0]
    nh = conv_2_w.shape[0]
    rh = res0_w3.shape[0]
    e = pre_vq_w.shape[0]
    lf = L // 4
    m = B * lf

    x4 = inputs.astype(_BF16).reshape(B, lf, 4).transpose(2, 0, 1).reshape(4, m)

    w1 = conv_1_w.astype(_F32)[:, 0, :]
    w1cat = jnp.zeros((2 * c1, 13), _F32)
    w1cat = w1cat.at[:c1, 3:7].set(w1)
    w1cat = w1cat.at[c1:, 5:9].set(w1)
    w1cat = w1cat.at[:c1, 12].set(conv_1_b.astype(_F32))
    w1cat = w1cat.at[c1:, 12].set(conv_1_b.astype(_F32))
    w1cat = w1cat.astype(_BF16)

    w2 = conv_2_w.astype(_F32)
    w2cat = jnp.concatenate(
        [w2[:, :, 1], w2[:, :, 2], w2[:, :, 0], w2[:, :, 3]],
        axis=1).astype(_BF16)
    b2 = conv_2_b.reshape(nh, 1).astype(_BF16)

    w3h = conv_3_w.astype(_F32)
    wh = jnp.concatenate(
        [w3h[:, :, 0], w3h[:, :, 1], w3h[:, :, 2],
         conv_3_b.reshape(nh, 1).astype(_F32)], axis=1).astype(_BF16)
    w3s = jnp.stack(
        [jnp.concatenate([w[:, :, 0], w[:, :, 1], w[:, :, 2]], axis=0)
         for w in (res0_w3.astype(_F32), res1_w3.astype(_F32))]
    ).astype(_BF16)
    w1s = jnp.stack([res0_w1[:, :, 0],
                     res1_w1[:, :, 0]]).astype(_BF16)
    wp = pre_vq_w[:, :, 0].astype(_BF16)
    bp = pre_vq_b.reshape(e, 1).astype(_F32)

    g = 4
    out = pl.pallas_call(
        _make_body(c1, rh, 2, g, lf),
        out_shape=jax.ShapeDtypeStruct((B, e, lf), _F32),
        grid_spec=pltpu.PrefetchScalarGridSpec(
            num_scalar_prefetch=0,
            grid=(B // g,),
            in_specs=[
                pl.BlockSpec((4, g * lf), lambda i: (0, i)),
                pl.BlockSpec((2 * c1, 13), lambda i: (0, 0)),
                pl.BlockSpec((nh, 4 * c1), lambda i: (0, 0)),
                pl.BlockSpec((nh, 1), lambda i: (0, 0)),
                pl.BlockSpec((nh, 3 * nh + 1), lambda i: (0, 0)),
                pl.BlockSpec((2, 3 * rh, nh), lambda i: (0, 0, 0)),
                pl.BlockSpec((2, nh, rh), lambda i: (0, 0, 0)),
                pl.BlockSpec((e, nh), lambda i: (0, 0)),
                pl.BlockSpec((e, 1), lambda i: (0, 0)),
            ],
            out_specs=pl.BlockSpec((g, e, lf), lambda i: (i, 0, 0)),
        ),
        compiler_params=pltpu.CompilerParams(
            dimension_semantics=("parallel",),
            vmem_limit_bytes=64 * 1024 * 1024,
        ),
    )(x4, w1cat, w2cat, b2, wh, w3s, w1s, wp, bp)
    return out
